# baseline (speedup 1.0000x reference)
import jax
import jax.numpy as jnp
from jax import lax
from jax.experimental import pallas as pl
from jax.experimental.pallas import tpu as pltpu
from jax.experimental.pallas import tpu_sc as plsc

SEQ = 2048
BATCH = 16
D = 64
V = 100000
PV = 2049
L = 16
NC = 2
NS = 16
NW = NC * NS
SCALE = 8.0
VECS = SEQ // L
REC = 2 * SEQ
UNROLL = 4
DEPTH = 3


def _emb_body(x_hbm, emb_hbm, pos_hbm, out_hbm,
              row_v, prow_v, c0, c1, c2, acc0, acc1,
              combo_hbm, sem_row, sem_in, sem_out):
    cid = lax.axis_index("c")
    sid = lax.axis_index("s")
    wid = cid * NS + sid

    cp_row = pltpu.async_copy(emb_hbm.at[wid], row_v, sem_row)
    cp_prow = pltpu.async_copy(pos_hbm.at[wid], prow_v, sem_row)

    pltpu.sync_copy(x_hbm.at[sid], c0.at[pl.ds(0, SEQ)])

    def scan_body(k, carry):
        v = c0[pl.ds(k * L, L)]
        m = jnp.minimum(v, 1)
        cs = plsc.cumsum(m)
        c0[pl.ds(SEQ + k * L, L)] = (cs + carry) * m
        return carry + jnp.max(cs)

    lax.fori_loop(0, VECS, scan_body, jnp.int32(0))
    pltpu.sync_copy(c0, combo_hbm.at[pl.ds(sid * REC, REC)])
    plsc.subcore_barrier()

    cbufs = (c0, c1, c2)
    abufs = (acc0, acc1)
    steps = [(r, b) for r in range(D // NW) for b in range(BATCH)]

    def prefetch(step, slot):
        _, b = step
        return pltpu.async_copy(combo_hbm.at[pl.ds(b * REC, REC)],
                                cbufs[slot], sem_in)

    pf = {i: prefetch(steps[i], i % DEPTH) for i in range(DEPTH)}
    store_h = {}
    row2 = None
    for i, (r, b) in enumerate(steps):
        p = i % 2
        d = wid + r * NW
        if i == 0:
            cp_row.wait()
            cp_prow.wait()
        elif r == 1 and b == 0:
            for h in row2:
                h.wait()
        pf.pop(i).wait()
        c_v = cbufs[i % DEPTH]
        acc_v = abufs[p]
        if p in store_h:
            store_h.pop(p).wait()

        def gat_body(k, _):
            for u in range(UNROLL):
                o = (k * UNROLL + u) * L
                tok = c_v[pl.ds(o, L)]
                pos = c_v[pl.ds(SEQ + o, L)]
                e = plsc.load_gather(row_v, [tok])
                pe = plsc.load_gather(prow_v, [pos])
                acc_v[pl.ds(o, L)] = e * SCALE + pe
            return 0

        lax.fori_loop(0, VECS // UNROLL, gat_body, 0)
        if r == 0 and b == BATCH - 1:
            row2 = (pltpu.async_copy(emb_hbm.at[wid + NW], row_v, sem_row),
                    pltpu.async_copy(pos_hbm.at[wid + NW], prow_v, sem_row))
        if i + DEPTH < len(steps):
            pf[i + DEPTH] = prefetch(steps[i + DEPTH], i % DEPTH)
        store_h[p] = pltpu.async_copy(acc_v, out_hbm.at[b, d], sem_out)
    for h in store_h.values():
        h.wait()


def kernel(x, emb_table, pos_table):
    x_t = x.T
    emb_t = emb_table.T
    pos_t = pos_table.T
    mesh = plsc.VectorSubcoreMesh(core_axis_name="c", subcore_axis_name="s")
    out_t = pl.kernel(
        _emb_body,
        out_type=jax.ShapeDtypeStruct((BATCH, D, SEQ), jnp.float32),
        mesh=mesh,
        compiler_params=pltpu.CompilerParams(
            use_tc_tiling_on_sc=True, needs_layout_passes=False),
        scratch_types=[
            pltpu.VMEM((V,), jnp.float32),
            pltpu.VMEM((PV,), jnp.float32),
            pltpu.VMEM((REC,), jnp.int32),
            pltpu.VMEM((REC,), jnp.int32),
            pltpu.VMEM((REC,), jnp.int32),
            pltpu.VMEM((SEQ,), jnp.float32),
            pltpu.VMEM((SEQ,), jnp.float32),
            pltpu.HBM((BATCH * REC,), jnp.int32),
            pltpu.SemaphoreType.DMA,
            pltpu.SemaphoreType.DMA,
            pltpu.SemaphoreType.DMA,
        ],
    )(x_t, emb_t, pos_t)
    return out_t.transpose(2, 0, 1)
